# baseline (speedup 1.0000x reference)
import functools

import jax
import jax.numpy as jnp
from jax import lax
from jax.experimental import pallas as pl
from jax.experimental.pallas import tpu as pltpu
from jax.experimental.pallas import tpu_sc as plsc

OUT_H = 7
OUT_W = 7
NBINS = OUT_H * OUT_W
SPATIAL_SCALE = 0.125
SN = 2
NTERMS = SN * SN * 4


def _coeff_kernel(roisT_ref, idx_ref, w_ref, *, B, H, W):
    f32 = jnp.float32
    npad = roisT_ref.shape[1]
    b = roisT_ref[0:1, :].astype(jnp.int32)
    cx = roisT_ref[1:2, :] * SPATIAL_SCALE
    cy = roisT_ref[2:3, :] * SPATIAL_SCALE
    rw = jnp.maximum(roisT_ref[3:4, :] * SPATIAL_SCALE, 1.0)
    rh = jnp.maximum(roisT_ref[4:5, :] * SPATIAL_SCALE, 1.0)
    theta = roisT_ref[5:6, :]
    cos_t = jnp.cos(theta)
    sin_t = jnp.sin(theta)
    bin_h = rh / OUT_H
    bin_w = rw / OUT_W
    bin_i = lax.broadcasted_iota(jnp.int32, (NBINS, npad), 0)
    oh = (bin_i // OUT_W).astype(f32)
    ow = (bin_i % OUT_W).astype(f32)
    for iy in range(SN):
        for ix in range(SN):
            yy = -rh * 0.5 + (oh + (iy + 0.5) / SN) * bin_h
            xx = -rw * 0.5 + (ow + (ix + 0.5) / SN) * bin_w
            x = xx * cos_t - yy * sin_t + cx
            y = xx * sin_t + yy * cos_t + cy
            valid = (y > -1.0) & (y < float(H)) & (x > -1.0) & (x < float(W))
            yc = jnp.clip(y, 0.0, float(H - 1))
            xc = jnp.clip(x, 0.0, float(W - 1))
            y0 = jnp.minimum(jnp.floor(yc).astype(jnp.int32), H - 2)
            x0 = jnp.minimum(jnp.floor(xc).astype(jnp.int32), W - 2)
            ly = yc - y0.astype(f32)
            lx = xc - x0.astype(f32)
            hy = 1.0 - ly
            hx = 1.0 - lx
            vf = valid.astype(f32) * (1.0 / (SN * SN))
            base = b * (H * W) + y0 * W + x0
            s = (iy * SN + ix) * 4
            idx_ref[s + 0] = base
            w_ref[s + 0] = hy * hx * vf
            idx_ref[s + 1] = base + 1
            w_ref[s + 1] = hy * lx * vf
            idx_ref[s + 2] = base + W
            w_ref[s + 2] = ly * hx * vf
            idx_ref[s + 3] = base + W + 1
            w_ref[s + 3] = ly * lx * vf


def _coeffs(rois, B, H, W):
    N = rois.shape[0]
    npad = ((N + 127) // 128) * 128
    roisT = jnp.zeros((8, npad), jnp.float32).at[:6, :N].set(rois.T)
    return pl.pallas_call(
        functools.partial(_coeff_kernel, B=B, H=H, W=W),
        out_shape=(
            jax.ShapeDtypeStruct((NTERMS, NBINS, npad), jnp.int32),
            jax.ShapeDtypeStruct((NTERMS, NBINS, npad), jnp.float32),
        ),
    )(roisT)


def _pack_words_kernel(x_ref, o_ref):
    xb = lax.bitcast_convert_type(x_ref[0], jnp.uint32)
    half = xb.shape[0] // 2
    lo = xb[:half]
    hi = xb[half:]
    lo_b = (lo + ((lo >> 16) & 1) + 0x7FFF) >> 16
    hi_b = ((hi + ((hi >> 16) & 1) + 0x7FFF) >> 16) << 16
    o_ref[0] = lax.bitcast_convert_type(lo_b | hi_b, jnp.int32)


def _pack_words(features):
    B, C, H, W = features.shape
    hw = H * W
    blk = 3200
    return pl.pallas_call(
        _pack_words_kernel,
        grid=(B, hw // blk),
        in_specs=[pl.BlockSpec((1, C, blk), lambda b, k: (b, 0, k))],
        out_specs=pl.BlockSpec((1, C // 2, blk), lambda b, k: (b, 0, k)),
        out_shape=jax.ShapeDtypeStruct((B, C // 2, hw), jnp.int32),
    )(features.reshape(B, C, hw))


def _make_sc_gather(R_pad, C, rpt, n_chunks, chunk):
    mesh = plsc.VectorSubcoreMesh(core_axis_name="c", subcore_axis_name="s")
    info = plsc.get_sparse_core_info()
    nc = info.num_cores
    idxc = chunk * NTERMS

    @functools.partial(
        pl.kernel,
        mesh=mesh,
        compiler_params=pltpu.CompilerParams(needs_layout_passes=False),
        out_type=jax.ShapeDtypeStruct((R_pad, C), jnp.float32),
        scratch_types=[
            pltpu.VMEM((2, idxc), jnp.int32),
            pltpu.VMEM((2, idxc), jnp.float32),
            pltpu.VMEM((2, idxc, C // 2), jnp.int32),
            pltpu.VMEM((2, chunk, C), jnp.float32),
            pltpu.SemaphoreType.DMA,
            pltpu.SemaphoreType.DMA,
            pltpu.SemaphoreType.DMA,
            pltpu.SemaphoreType.DMA,
            pltpu.SemaphoreType.DMA,
            pltpu.SemaphoreType.DMA,
            pltpu.SemaphoreType.DMA,
            pltpu.SemaphoreType.DMA,
        ],
    )
    def sc_gather(feat_hbm, idx_hbm, w_hbm, out_hbm, idx_v, w_v, gath_v, ost_v,
                  s_i0, s_i1, s_w0, s_w1, s_g0, s_g1, s_o0, s_o1):
        s_i = (s_i0, s_i1)
        s_w = (s_w0, s_w1)
        s_g = (s_g0, s_g1)
        s_o = (s_o0, s_o1)
        wid = lax.axis_index("s") * nc + lax.axis_index("c")
        gbase = wid * n_chunks
        row0 = wid * rpt

        pltpu.async_copy(idx_hbm.at[gbase], idx_v.at[0], s_i[0]).wait()
        pltpu.async_copy(w_hbm.at[gbase], w_v.at[0], s_w[0]).wait()
        pltpu.async_copy(idx_hbm.at[gbase + 1], idx_v.at[1], s_i[1])
        pltpu.async_copy(w_hbm.at[gbase + 1], w_v.at[1], s_w[1])
        pltpu.async_copy(feat_hbm.at[idx_v.at[0]], gath_v.at[0], s_g[0])

        def pair_body(j, carry):
            for b in (0, 1):
                g = 2 * j + b
                @pl.when(g + 1 < n_chunks)
                def _():
                    pltpu.make_async_copy(
                        idx_hbm.at[gbase], idx_v.at[1 - b], s_i[1 - b]).wait()
                    pltpu.make_async_copy(
                        w_hbm.at[gbase], w_v.at[1 - b], s_w[1 - b]).wait()
                    pltpu.async_copy(
                        feat_hbm.at[idx_v.at[1 - b]], gath_v.at[1 - b], s_g[1 - b])
                pltpu.make_async_copy(
                    feat_hbm.at[idx_v.at[b]], gath_v.at[b], s_g[b]).wait()
                @pl.when(g >= 2)
                def _():
                    pltpu.make_async_copy(
                        ost_v.at[b], out_hbm.at[pl.ds(row0, chunk)], s_o[b]).wait()

                def row_body(r, carry2):
                    w16 = w_v[b, pl.ds(r * NTERMS, NTERMS)]
                    wbs = [w16[i] for i in range(NTERMS)]
                    for cc in range(C // 32):
                        pe, po = [], []
                        for i in range(NTERMS):
                            g32 = plsc.bitcast(
                                gath_v[b, r * NTERMS + i, pl.ds(cc * 16, 16)],
                                jnp.bfloat16)
                            ge, go = plsc.unpack(
                                g32, format=plsc.PackFormat.INTERLEAVED)
                            pe.append(wbs[i] * ge)
                            po.append(wbs[i] * go)
                        while len(pe) > 1:
                            pe = [pe[2 * k] + pe[2 * k + 1] for k in range(len(pe) // 2)]
                            po = [po[2 * k] + po[2 * k + 1] for k in range(len(po) // 2)]
                        ost_v[b, r, pl.ds(cc * 16, 16)] = pe[0]
                        ost_v[b, r, pl.ds(C // 2 + cc * 16, 16)] = po[0]
                    return carry2

                lax.fori_loop(0, chunk, row_body, 0)
                pltpu.async_copy(
                    ost_v.at[b], out_hbm.at[pl.ds(row0 + g * chunk, chunk)], s_o[b])
                @pl.when(g + 2 < n_chunks)
                def _():
                    pltpu.async_copy(idx_hbm.at[gbase + g + 2], idx_v.at[b], s_i[b])
                    pltpu.async_copy(w_hbm.at[gbase + g + 2], w_v.at[b], s_w[b])
            return carry

        lax.fori_loop(0, n_chunks // 2, pair_body, 0)
        for b in (0, 1):
            pltpu.make_async_copy(
                ost_v.at[b], out_hbm.at[pl.ds(row0, chunk)], s_o[b]).wait()

    return sc_gather


def kernel(features, rois):
    B, C, H, W = features.shape
    N = rois.shape[0]
    R = N * NBINS
    tiles = 32
    chunk = 8
    rpt = ((R + tiles * chunk - 1) // (tiles * chunk)) * chunk
    R_pad = tiles * rpt
    n_chunks = rpt // chunk

    idx3, w3 = _coeffs(rois, B, H, W)
    idx_rt = jnp.transpose(idx3[:, :, :N], (2, 1, 0)).reshape(R * NTERMS)
    w_rt = jnp.transpose(w3[:, :, :N], (2, 1, 0)).reshape(R * NTERMS)
    idxf = jnp.zeros((R_pad * NTERMS,), jnp.int32).at[: R * NTERMS].set(idx_rt)
    wf = jnp.zeros((R_pad * NTERMS,), jnp.float32).at[: R * NTERMS].set(w_rt)
    idxc = chunk * NTERMS
    feat_words = (_pack_words(features)
                  .transpose(0, 2, 1).reshape(B * H * W, C // 2))

    sc = _make_sc_gather(R_pad, C, rpt, n_chunks, chunk)
    out_rows = sc(feat_words, idxf.reshape(-1, idxc), wf.reshape(-1, idxc))
    return out_rows[:R].reshape(N, NBINS, C).transpose(0, 2, 1).reshape(N, C, OUT_H, OUT_W)


# speedup vs baseline: 1.1618x; 1.1618x over previous
import functools

import jax
import jax.numpy as jnp
from jax import lax
from jax.experimental import pallas as pl
from jax.experimental.pallas import tpu as pltpu
from jax.experimental.pallas import tpu_sc as plsc

OUT_H = 7
OUT_W = 7
NBINS = OUT_H * OUT_W
SPATIAL_SCALE = 0.125
SN = 2
NTERMS = SN * SN * 4


def _coeff_kernel(roisT_ref, idx_ref, w_ref, *, B, H, W):
    f32 = jnp.float32
    npad = roisT_ref.shape[1]
    b = roisT_ref[0:1, :].astype(jnp.int32)
    cx = roisT_ref[1:2, :] * SPATIAL_SCALE
    cy = roisT_ref[2:3, :] * SPATIAL_SCALE
    rw = jnp.maximum(roisT_ref[3:4, :] * SPATIAL_SCALE, 1.0)
    rh = jnp.maximum(roisT_ref[4:5, :] * SPATIAL_SCALE, 1.0)
    theta = roisT_ref[5:6, :]
    cos_t = jnp.cos(theta)
    sin_t = jnp.sin(theta)
    bin_h = rh / OUT_H
    bin_w = rw / OUT_W
    bin_i = lax.broadcasted_iota(jnp.int32, (NBINS, npad), 0)
    oh = (bin_i // OUT_W).astype(f32)
    ow = (bin_i % OUT_W).astype(f32)
    for iy in range(SN):
        for ix in range(SN):
            yy = -rh * 0.5 + (oh + (iy + 0.5) / SN) * bin_h
            xx = -rw * 0.5 + (ow + (ix + 0.5) / SN) * bin_w
            x = xx * cos_t - yy * sin_t + cx
            y = xx * sin_t + yy * cos_t + cy
            valid = (y > -1.0) & (y < float(H)) & (x > -1.0) & (x < float(W))
            yc = jnp.clip(y, 0.0, float(H - 1))
            xc = jnp.clip(x, 0.0, float(W - 1))
            y0 = jnp.minimum(jnp.floor(yc).astype(jnp.int32), H - 2)
            x0 = jnp.minimum(jnp.floor(xc).astype(jnp.int32), W - 2)
            ly = yc - y0.astype(f32)
            lx = xc - x0.astype(f32)
            hy = 1.0 - ly
            hx = 1.0 - lx
            vf = valid.astype(f32) * (1.0 / (SN * SN))
            base = b * (H * W) + y0 * W + x0
            s = (iy * SN + ix) * 4
            idx_ref[s + 0] = base
            w_ref[s + 0] = hy * hx * vf
            idx_ref[s + 1] = base + 1
            w_ref[s + 1] = hy * lx * vf
            idx_ref[s + 2] = base + W
            w_ref[s + 2] = ly * hx * vf
            idx_ref[s + 3] = base + W + 1
            w_ref[s + 3] = ly * lx * vf


def _coeffs(rois, B, H, W):
    N = rois.shape[0]
    npad = ((N + 127) // 128) * 128
    roisT = jnp.zeros((8, npad), jnp.float32).at[:6, :N].set(rois.T)
    return pl.pallas_call(
        functools.partial(_coeff_kernel, B=B, H=H, W=W),
        out_shape=(
            jax.ShapeDtypeStruct((NTERMS, NBINS, npad), jnp.int32),
            jax.ShapeDtypeStruct((NTERMS, NBINS, npad), jnp.float32),
        ),
    )(roisT)


def _pack_words_kernel(x_ref, o_ref):
    xb = lax.bitcast_convert_type(x_ref[...], jnp.uint32)
    half = xb.shape[1] // 2
    lo = xb[:, :half]
    hi = xb[:, half:]
    lo_b = (lo + ((lo >> 16) & 1) + 0x7FFF) >> 16
    hi_b = ((hi + ((hi >> 16) & 1) + 0x7FFF) >> 16) << 16
    o_ref[...] = lax.bitcast_convert_type(lo_b | hi_b, jnp.int32)


def _pack_words(feat_rows):
    nrows, C_ = feat_rows.shape
    blk = 2048
    return pl.pallas_call(
        _pack_words_kernel,
        grid=(nrows // blk,),
        in_specs=[pl.BlockSpec((blk, C_), lambda k: (k, 0))],
        out_specs=pl.BlockSpec((blk, C_ // 2), lambda k: (k, 0)),
        out_shape=jax.ShapeDtypeStruct((nrows, C_ // 2), jnp.int32),
    )(feat_rows)


def _make_sc_gather(R_pad, C, rpt, n_chunks, chunk):
    mesh = plsc.VectorSubcoreMesh(core_axis_name="c", subcore_axis_name="s")
    info = plsc.get_sparse_core_info()
    nc = info.num_cores
    idxc = chunk * NTERMS

    @functools.partial(
        pl.kernel,
        mesh=mesh,
        compiler_params=pltpu.CompilerParams(needs_layout_passes=False),
        out_type=jax.ShapeDtypeStruct((R_pad, C), jnp.float32),
        scratch_types=[
            pltpu.VMEM((2, idxc), jnp.int32),
            pltpu.VMEM((2, idxc), jnp.float32),
            pltpu.VMEM((2, idxc, C // 2), jnp.int32),
            pltpu.VMEM((2, chunk, C), jnp.float32),
            pltpu.SemaphoreType.DMA,
            pltpu.SemaphoreType.DMA,
            pltpu.SemaphoreType.DMA,
            pltpu.SemaphoreType.DMA,
            pltpu.SemaphoreType.DMA,
            pltpu.SemaphoreType.DMA,
            pltpu.SemaphoreType.DMA,
            pltpu.SemaphoreType.DMA,
        ],
    )
    def sc_gather(feat_hbm, idx_hbm, w_hbm, out_hbm, idx_v, w_v, gath_v, ost_v,
                  s_i0, s_i1, s_w0, s_w1, s_g0, s_g1, s_o0, s_o1):
        s_i = (s_i0, s_i1)
        s_w = (s_w0, s_w1)
        s_g = (s_g0, s_g1)
        s_o = (s_o0, s_o1)
        wid = lax.axis_index("s") * nc + lax.axis_index("c")
        gbase = wid * n_chunks
        row0 = wid * rpt

        pltpu.async_copy(idx_hbm.at[gbase], idx_v.at[0], s_i[0]).wait()
        pltpu.async_copy(w_hbm.at[gbase], w_v.at[0], s_w[0]).wait()
        pltpu.async_copy(idx_hbm.at[gbase + 1], idx_v.at[1], s_i[1])
        pltpu.async_copy(w_hbm.at[gbase + 1], w_v.at[1], s_w[1])
        pltpu.async_copy(feat_hbm.at[idx_v.at[0]], gath_v.at[0], s_g[0])

        def pair_body(j, carry):
            for b in (0, 1):
                g = 2 * j + b
                @pl.when(g + 1 < n_chunks)
                def _():
                    pltpu.make_async_copy(
                        idx_hbm.at[gbase], idx_v.at[1 - b], s_i[1 - b]).wait()
                    pltpu.make_async_copy(
                        w_hbm.at[gbase], w_v.at[1 - b], s_w[1 - b]).wait()
                    pltpu.async_copy(
                        feat_hbm.at[idx_v.at[1 - b]], gath_v.at[1 - b], s_g[1 - b])
                pltpu.make_async_copy(
                    feat_hbm.at[idx_v.at[b]], gath_v.at[b], s_g[b]).wait()
                @pl.when(g >= 2)
                def _():
                    pltpu.make_async_copy(
                        ost_v.at[b], out_hbm.at[pl.ds(row0, chunk)], s_o[b]).wait()

                def row_body(r, carry2):
                    w16 = w_v[b, pl.ds(r * NTERMS, NTERMS)]
                    wpk = []
                    for i in range(NTERMS):
                        sv = jnp.full((16,), w16[i], jnp.float32)
                        wpk.append(plsc.pack(sv, sv,
                                             format=plsc.PackFormat.INTERLEAVED))
                    for cc in range(C // 32):
                        ps = [
                            wpk[i] * plsc.bitcast(
                                gath_v[b, r * NTERMS + i, pl.ds(cc * 16, 16)],
                                jnp.bfloat16)
                            for i in range(NTERMS)
                        ]
                        while len(ps) > 1:
                            ps = [ps[2 * k] + ps[2 * k + 1] for k in range(len(ps) // 2)]
                        te, to = plsc.unpack(ps[0], format=plsc.PackFormat.INTERLEAVED)
                        ost_v[b, r, pl.ds(cc * 16, 16)] = te
                        ost_v[b, r, pl.ds(C // 2 + cc * 16, 16)] = to
                    return carry2

                lax.fori_loop(0, chunk, row_body, 0)
                pltpu.async_copy(
                    ost_v.at[b], out_hbm.at[pl.ds(row0 + g * chunk, chunk)], s_o[b])
                @pl.when(g + 2 < n_chunks)
                def _():
                    pltpu.async_copy(idx_hbm.at[gbase + g + 2], idx_v.at[b], s_i[b])
                    pltpu.async_copy(w_hbm.at[gbase + g + 2], w_v.at[b], s_w[b])
            return carry

        lax.fori_loop(0, n_chunks // 2, pair_body, 0)
        for b in (0, 1):
            pltpu.make_async_copy(
                ost_v.at[b], out_hbm.at[pl.ds(row0, chunk)], s_o[b]).wait()

    return sc_gather


def kernel(features, rois):
    B, C, H, W = features.shape
    N = rois.shape[0]
    R = N * NBINS
    tiles = 32
    chunk = 8
    rpt = ((R + tiles * chunk - 1) // (tiles * chunk)) * chunk
    R_pad = tiles * rpt
    n_chunks = rpt // chunk

    idx3, w3 = _coeffs(rois, B, H, W)
    idx_rt = jnp.transpose(idx3[:, :, :N], (2, 1, 0)).reshape(R * NTERMS)
    w_rt = jnp.transpose(w3[:, :, :N], (2, 1, 0)).reshape(R * NTERMS)
    idxf = jnp.zeros((R_pad * NTERMS,), jnp.int32).at[: R * NTERMS].set(idx_rt)
    wf = jnp.zeros((R_pad * NTERMS,), jnp.float32).at[: R * NTERMS].set(w_rt)
    idxc = chunk * NTERMS
    feat_rows = features.transpose(0, 2, 3, 1).reshape(B * H * W, C)
    feat_words = _pack_words(feat_rows)

    sc = _make_sc_gather(R_pad, C, rpt, n_chunks, chunk)
    out_rows = sc(feat_words, idxf.reshape(-1, idxc), wf.reshape(-1, idxc))
    return out_rows[:R].reshape(N, NBINS, C).transpose(0, 2, 1).reshape(N, C, OUT_H, OUT_W)


# speedup vs baseline: 1.2288x; 1.0576x over previous
import functools

import jax
import jax.numpy as jnp
from jax import lax
from jax.experimental import pallas as pl
from jax.experimental.pallas import tpu as pltpu
from jax.experimental.pallas import tpu_sc as plsc

OUT_H = 7
OUT_W = 7
NBINS = OUT_H * OUT_W
SPATIAL_SCALE = 0.125
SN = 2
NTERMS = SN * SN * 4


def _coeff_kernel(roisT_ref, idx_ref, w_ref, *, B, H, W):
    f32 = jnp.float32
    npad = roisT_ref.shape[1]
    b = roisT_ref[0:1, :].astype(jnp.int32)
    cx = roisT_ref[1:2, :] * SPATIAL_SCALE
    cy = roisT_ref[2:3, :] * SPATIAL_SCALE
    rw = jnp.maximum(roisT_ref[3:4, :] * SPATIAL_SCALE, 1.0)
    rh = jnp.maximum(roisT_ref[4:5, :] * SPATIAL_SCALE, 1.0)
    theta = roisT_ref[5:6, :]
    cos_t = jnp.cos(theta)
    sin_t = jnp.sin(theta)
    bin_h = rh / OUT_H
    bin_w = rw / OUT_W
    bin_i = lax.broadcasted_iota(jnp.int32, (NBINS, npad), 0)
    oh = (bin_i // OUT_W).astype(f32)
    ow = (bin_i % OUT_W).astype(f32)
    for iy in range(SN):
        for ix in range(SN):
            yy = -rh * 0.5 + (oh + (iy + 0.5) / SN) * bin_h
            xx = -rw * 0.5 + (ow + (ix + 0.5) / SN) * bin_w
            x = xx * cos_t - yy * sin_t + cx
            y = xx * sin_t + yy * cos_t + cy
            valid = (y > -1.0) & (y < float(H)) & (x > -1.0) & (x < float(W))
            yc = jnp.clip(y, 0.0, float(H - 1))
            xc = jnp.clip(x, 0.0, float(W - 1))
            y0 = jnp.minimum(jnp.floor(yc).astype(jnp.int32), H - 2)
            x0 = jnp.minimum(jnp.floor(xc).astype(jnp.int32), W - 2)
            ly = yc - y0.astype(f32)
            lx = xc - x0.astype(f32)
            hy = 1.0 - ly
            hx = 1.0 - lx
            vf = valid.astype(f32) * (1.0 / (SN * SN))
            base = b * (H * W) + y0 * W + x0
            s = (iy * SN + ix) * 4
            idx_ref[s + 0] = base
            w_ref[s + 0] = hy * hx * vf
            idx_ref[s + 1] = base + 1
            w_ref[s + 1] = hy * lx * vf
            idx_ref[s + 2] = base + W
            w_ref[s + 2] = ly * hx * vf
            idx_ref[s + 3] = base + W + 1
            w_ref[s + 3] = ly * lx * vf


def _coeffs(rois, B, H, W):
    N = rois.shape[0]
    npad = ((N + 127) // 128) * 128
    roisT = jnp.zeros((8, npad), jnp.float32).at[:6, :N].set(rois.T)
    return pl.pallas_call(
        functools.partial(_coeff_kernel, B=B, H=H, W=W),
        out_shape=(
            jax.ShapeDtypeStruct((NTERMS, NBINS, npad), jnp.int32),
            jax.ShapeDtypeStruct((NTERMS, NBINS, npad), jnp.float32),
        ),
    )(roisT)


def _pack_words_kernel(x_ref, o_ref):
    xb = lax.bitcast_convert_type(x_ref[...], jnp.uint32)
    half = xb.shape[1] // 2
    lo = xb[:, :half]
    hi = xb[:, half:]
    lo_b = (lo + ((lo >> 16) & 1) + 0x7FFF) >> 16
    hi_b = ((hi + ((hi >> 16) & 1) + 0x7FFF) >> 16) << 16
    o_ref[...] = lax.bitcast_convert_type(lo_b | hi_b, jnp.int32)


def _pack_words(feat_rows):
    nrows, C_ = feat_rows.shape
    blk = 2048
    return pl.pallas_call(
        _pack_words_kernel,
        grid=(nrows // blk,),
        in_specs=[pl.BlockSpec((blk, C_), lambda k: (k, 0))],
        out_specs=pl.BlockSpec((blk, C_ // 2), lambda k: (k, 0)),
        out_shape=jax.ShapeDtypeStruct((nrows, C_ // 2), jnp.int32),
    )(feat_rows)


def _make_sc_gather(R_pad, C, rpt, n_chunks, chunk):
    mesh = plsc.VectorSubcoreMesh(core_axis_name="c", subcore_axis_name="s")
    info = plsc.get_sparse_core_info()
    nc = info.num_cores
    idxc = chunk * NTERMS

    @functools.partial(
        pl.kernel,
        mesh=mesh,
        compiler_params=pltpu.CompilerParams(needs_layout_passes=False),
        out_type=jax.ShapeDtypeStruct((R_pad, C), jnp.float32),
        scratch_types=[
            pltpu.VMEM((2, idxc), jnp.int32),
            pltpu.VMEM((2, idxc), jnp.float32),
            pltpu.VMEM((2, idxc, C // 2), jnp.int32),
            pltpu.VMEM((2, chunk, C), jnp.float32),
            pltpu.SemaphoreType.DMA,
            pltpu.SemaphoreType.DMA,
            pltpu.SemaphoreType.DMA,
            pltpu.SemaphoreType.DMA,
            pltpu.SemaphoreType.DMA,
            pltpu.SemaphoreType.DMA,
            pltpu.SemaphoreType.DMA,
            pltpu.SemaphoreType.DMA,
        ],
    )
    def sc_gather(feat_hbm, idx_hbm, w_hbm, out_hbm, idx_v, w_v, gath_v, ost_v,
                  s_i0, s_i1, s_w0, s_w1, s_g0, s_g1, s_o0, s_o1):
        s_i = (s_i0, s_i1)
        s_w = (s_w0, s_w1)
        s_g = (s_g0, s_g1)
        s_o = (s_o0, s_o1)
        wid = lax.axis_index("s") * nc + lax.axis_index("c")
        gbase = wid * n_chunks
        row0 = wid * rpt

        pltpu.async_copy(idx_hbm.at[gbase], idx_v.at[0], s_i[0]).wait()
        pltpu.async_copy(w_hbm.at[gbase], w_v.at[0], s_w[0]).wait()
        pltpu.async_copy(idx_hbm.at[gbase + 1], idx_v.at[1], s_i[1])
        pltpu.async_copy(w_hbm.at[gbase + 1], w_v.at[1], s_w[1])
        pltpu.async_copy(feat_hbm.at[idx_v.at[0]], gath_v.at[0], s_g[0])

        def pair_body(j, carry):
            for b in (0, 1):
                g = 2 * j + b
                @pl.when(g + 1 < n_chunks)
                def _():
                    pltpu.make_async_copy(
                        idx_hbm.at[gbase], idx_v.at[1 - b], s_i[1 - b]).wait()
                    pltpu.make_async_copy(
                        w_hbm.at[gbase], w_v.at[1 - b], s_w[1 - b]).wait()
                    pltpu.async_copy(
                        feat_hbm.at[idx_v.at[1 - b]], gath_v.at[1 - b], s_g[1 - b])
                pltpu.make_async_copy(
                    feat_hbm.at[idx_v.at[b]], gath_v.at[b], s_g[b]).wait()
                @pl.when(g >= 2)
                def _():
                    pltpu.make_async_copy(
                        ost_v.at[b], out_hbm.at[pl.ds(row0, chunk)], s_o[b]).wait()

                @plsc.parallel_loop(0, chunk, step=1, unroll=2)
                def row_body(r):
                    w16 = w_v[b, pl.ds(r * NTERMS, NTERMS)]
                    wpk = []
                    for i in range(NTERMS):
                        sv = jnp.full((16,), w16[i], jnp.float32)
                        wpk.append(plsc.pack(sv, sv,
                                             format=plsc.PackFormat.INTERLEAVED))
                    for cc in range(C // 32):
                        ps = [
                            wpk[i] * plsc.bitcast(
                                gath_v[b, r * NTERMS + i, pl.ds(cc * 16, 16)],
                                jnp.bfloat16)
                            for i in range(NTERMS)
                        ]
                        while len(ps) > 1:
                            ps = [ps[2 * k] + ps[2 * k + 1] for k in range(len(ps) // 2)]
                        te, to = plsc.unpack(ps[0], format=plsc.PackFormat.INTERLEAVED)
                        ost_v[b, r, pl.ds(cc * 16, 16)] = te
                        ost_v[b, r, pl.ds(C // 2 + cc * 16, 16)] = to
                pltpu.async_copy(
                    ost_v.at[b], out_hbm.at[pl.ds(row0 + g * chunk, chunk)], s_o[b])
                @pl.when(g + 2 < n_chunks)
                def _():
                    pltpu.async_copy(idx_hbm.at[gbase + g + 2], idx_v.at[b], s_i[b])
                    pltpu.async_copy(w_hbm.at[gbase + g + 2], w_v.at[b], s_w[b])
            return carry

        lax.fori_loop(0, n_chunks // 2, pair_body, 0)
        for b in (0, 1):
            pltpu.make_async_copy(
                ost_v.at[b], out_hbm.at[pl.ds(row0, chunk)], s_o[b]).wait()

    return sc_gather


def kernel(features, rois):
    B, C, H, W = features.shape
    N = rois.shape[0]
    R = N * NBINS
    tiles = 32
    chunk = 8
    rpt = ((R + tiles * chunk - 1) // (tiles * chunk)) * chunk
    R_pad = tiles * rpt
    n_chunks = rpt // chunk

    idx3, w3 = _coeffs(rois, B, H, W)
    idx_rt = jnp.transpose(idx3[:, :, :N], (2, 1, 0)).reshape(R * NTERMS)
    w_rt = jnp.transpose(w3[:, :, :N], (2, 1, 0)).reshape(R * NTERMS)
    idxf = jnp.zeros((R_pad * NTERMS,), jnp.int32).at[: R * NTERMS].set(idx_rt)
    wf = jnp.zeros((R_pad * NTERMS,), jnp.float32).at[: R * NTERMS].set(w_rt)
    idxc = chunk * NTERMS
    feat_rows = features.transpose(0, 2, 3, 1).reshape(B * H * W, C)
    feat_words = _pack_words(feat_rows)

    sc = _make_sc_gather(R_pad, C, rpt, n_chunks, chunk)
    out_rows = sc(feat_words, idxf.reshape(-1, idxc), wf.reshape(-1, idxc))
    return out_rows[:R].reshape(N, NBINS, C).transpose(0, 2, 1).reshape(N, C, OUT_H, OUT_W)
